# baseline (speedup 1.0000x reference)
import functools

import jax
import jax.numpy as jnp
from jax import lax
from jax.experimental import pallas as pl
from jax.experimental.pallas import tpu as pltpu
from jax.experimental.pallas import tpu_sc as plsc

N = 10000
E = 320000
D = 128
H = 256
C = 40
K_PROP = 3

NC = 2
NS = 16
NW = NC * NS
CHUNK = 128
CPW = 80
GPG = 8
NCHUNKS = NW * CPW
EPAD = NCHUNKS * CHUNK
NACC = 10240
RPS = NACC // NS

_mesh = plsc.VectorSubcoreMesh(
    core_axis_name="c", subcore_axis_name="s", num_cores=NC, num_subcores=NS)


@functools.partial(
    pl.kernel,
    out_type=jax.ShapeDtypeStruct((NC, NACC), jnp.float32),
    mesh=_mesh,
    scratch_types=[
        pltpu.VMEM((CPW, CHUNK), jnp.int32),
        pltpu.VMEM((CHUNK,), jnp.float32),
        pltpu.VMEM((RPS,), jnp.float32),
        pltpu.VMEM_SHARED((NACC,), jnp.float32),
    ],
)
def _sc_degrees(dstc_hbm, out_hbm, didx, ones, zbuf, acc):
    c = lax.axis_index("c")
    s = lax.axis_index("s")
    w = c * NS + s
    one16 = jnp.ones((16,), jnp.float32)
    zero16 = jnp.zeros((16,), jnp.float32)
    for l in range(CHUNK // 16):
        ones[pl.ds(16 * l, 16)] = one16

    def zfill(i, _):
        zbuf[pl.ds(16 * i, 16)] = zero16
        return 0
    lax.fori_loop(0, RPS // 16, zfill, 0)
    pltpu.sync_copy(zbuf, acc.at[pl.ds(s * RPS, RPS)])
    plsc.subcore_barrier()

    pltpu.sync_copy(dstc_hbm.at[w], didx)

    def body(j, _):
        pltpu.sync_copy(ones, acc.at[didx.at[j]], add=True)
        return 0
    lax.fori_loop(0, CPW, body, 0)
    plsc.subcore_barrier()
    pltpu.sync_copy(acc.at[pl.ds(s * RPS, RPS)],
                    out_hbm.at[c, pl.ds(s * RPS, RPS)])


@functools.partial(
    pl.kernel,
    out_type=jax.ShapeDtypeStruct((NC, NACC, D), jnp.float32),
    mesh=_mesh,
    scratch_types=[
        pltpu.VMEM((CPW, CHUNK), jnp.int32),
        pltpu.VMEM((CPW, CHUNK), jnp.int32),
        pltpu.VMEM((CHUNK, D), jnp.float32),
        pltpu.VMEM((8, D), jnp.float32),
        pltpu.VMEM((768,), jnp.float32),
        pltpu.VMEM_SHARED((NACC, D), jnp.float32),
        pltpu.SemaphoreType.DMA,
    ],
)
def _sc_edge_pass(z_hbm, srcc_hbm, dstc_hbm, out_hbm,
                  sidx, didx, rows, zrow, _stag, acc, sem):
    c = lax.axis_index("c")
    s = lax.axis_index("s")
    w = c * NS + s
    zero16 = jnp.zeros((16,), jnp.float32)
    for r in range(8):
        for l in range(D // 16):
            zrow[r, pl.ds(16 * l, 16)] = zero16

    def zfill(i, _):
        pltpu.sync_copy(zrow, acc.at[pl.ds(s * RPS + 8 * i, 8)])
        return 0
    lax.fori_loop(0, RPS // 8, zfill, 0)
    plsc.subcore_barrier()

    pltpu.sync_copy(srcc_hbm.at[w], sidx)
    pltpu.sync_copy(dstc_hbm.at[w], didx)

    def body(j, _):
        pltpu.async_copy(z_hbm.at[sidx.at[j]], rows, sem).wait()
        pltpu.sync_copy(rows, acc.at[didx.at[j]], add=True)
        return 0
    lax.fori_loop(0, CPW, body, 0)
    plsc.subcore_barrier()
    pltpu.sync_copy(acc.at[pl.ds(s * RPS, RPS)],
                    out_hbm.at[c, pl.ds(s * RPS, RPS)])


def _scale_body(x_ref, s_ref, o_ref, ys_ref):
    z = x_ref[...] * s_ref[...]
    o_ref[...] = z
    ys_ref[...] = z


def _combine_body(p0_ref, p1_ref, n2_ref, ys_ref, z_ref, yso_ref):
    z = (p0_ref[...] + p1_ref[...]) * n2_ref[...]
    z_ref[...] = z
    yso_ref[...] = ys_ref[...] + z


def _mlp_body(ys_ref, sq_ref, w1_ref, b1_ref, w2_ref, b2_ref, o_ref):
    y = ys_ref[...] * (sq_ref[...] * (1.0 / (K_PROP + 1)))
    h1 = jnp.maximum(
        jnp.dot(y, w1_ref[...], preferred_element_type=jnp.float32)
        + b1_ref[...], 0.0)
    logits = (jnp.dot(h1, w2_ref[...], preferred_element_type=jnp.float32)
              + b2_ref[...])
    m = jnp.max(logits, axis=-1, keepdims=True)
    lse = jnp.log(jnp.sum(jnp.exp(logits - m), axis=-1, keepdims=True))
    o_ref[...] = logits - m - lse


_RB = 1024


def _tc_scale(x, scal):
    return pl.pallas_call(
        _scale_body,
        grid=(NACC // _RB,),
        in_specs=[pl.BlockSpec((_RB, D), lambda i: (i, 0)),
                  pl.BlockSpec((_RB, 1), lambda i: (i, 0))],
        out_specs=[pl.BlockSpec((_RB, D), lambda i: (i, 0)),
                   pl.BlockSpec((_RB, D), lambda i: (i, 0))],
        out_shape=[jax.ShapeDtypeStruct((NACC, D), jnp.float32),
                   jax.ShapeDtypeStruct((NACC, D), jnp.float32)],
    )(x, scal)


def _tc_combine(p0, p1, n2, ysum):
    return pl.pallas_call(
        _combine_body,
        grid=(NACC // _RB,),
        in_specs=[pl.BlockSpec((_RB, D), lambda i: (i, 0)),
                  pl.BlockSpec((_RB, D), lambda i: (i, 0)),
                  pl.BlockSpec((_RB, 1), lambda i: (i, 0)),
                  pl.BlockSpec((_RB, D), lambda i: (i, 0))],
        out_specs=[pl.BlockSpec((_RB, D), lambda i: (i, 0)),
                   pl.BlockSpec((_RB, D), lambda i: (i, 0))],
        out_shape=[jax.ShapeDtypeStruct((NACC, D), jnp.float32),
                   jax.ShapeDtypeStruct((NACC, D), jnp.float32)],
    )(p0, p1, n2, ysum)


_MB = 1000


def _tc_mlp(ysum, sq, W1, b1, W2, b2):
    return pl.pallas_call(
        _mlp_body,
        grid=(N // _MB,),
        in_specs=[pl.BlockSpec((_MB, D), lambda i: (i, 0)),
                  pl.BlockSpec((_MB, 1), lambda i: (i, 0)),
                  pl.BlockSpec((D, H), lambda i: (0, 0)),
                  pl.BlockSpec((1, H), lambda i: (0, 0)),
                  pl.BlockSpec((H, C), lambda i: (0, 0)),
                  pl.BlockSpec((1, C), lambda i: (0, 0))],
        out_specs=pl.BlockSpec((_MB, C), lambda i: (i, 0)),
        out_shape=jax.ShapeDtypeStruct((N, C), jnp.float32),
    )(ysum, sq, W1, b1.reshape(1, H), W2, b2.reshape(1, C))


def kernel(feats, edge_index, W1, b1, W2, b2):
    src = edge_index[0]
    dst = edge_index[1]
    pad = EPAD - E
    srcc = jnp.concatenate(
        [src, jnp.zeros((pad,), jnp.int32)]).reshape(NW, CPW, CHUNK)
    pad_dst = jnp.full((pad,), N, jnp.int32)
    dstc = jnp.concatenate([dst, pad_dst]).reshape(NW, CPW, CHUNK)
    featsp = jnp.concatenate(
        [feats, jnp.zeros((NACC - N, D), jnp.float32)], axis=0)

    deg_part = _sc_degrees(dstc)
    deg = jnp.maximum(deg_part[0] + deg_part[1], 1.0)
    n1 = lax.rsqrt(deg)[:, None]
    n2 = (1.0 / deg)[:, None]
    sq = jnp.sqrt(deg)[:, None]

    z, ysum = _tc_scale(featsp, n1)
    for _ in range(K_PROP):
        p = _sc_edge_pass(z, srcc, dstc)
        z, ysum = _tc_combine(p[0], p[1], n2, ysum)

    return _tc_mlp(ysum, sq, W1, b1, W2, b2)


# speedup vs baseline: 1.0003x; 1.0003x over previous
import functools

import jax
import jax.numpy as jnp
from jax import lax
from jax.experimental import pallas as pl
from jax.experimental.pallas import tpu as pltpu
from jax.experimental.pallas import tpu_sc as plsc

N = 10000
E = 320000
D = 128
H = 256
C = 40
K_PROP = 3

NC = 2
NS = 16
NW = NC * NS
CHUNK = 128
CPW = 80
GPG = 8
NCHUNKS = NW * CPW
EPAD = NCHUNKS * CHUNK
NACC = 10240
RPS = NACC // NS

_mesh = plsc.VectorSubcoreMesh(
    core_axis_name="c", subcore_axis_name="s", num_cores=NC, num_subcores=NS)


@functools.partial(
    pl.kernel,
    out_type=jax.ShapeDtypeStruct((NC, NACC), jnp.float32),
    mesh=_mesh,
    scratch_types=[
        pltpu.VMEM((CPW, CHUNK), jnp.int32),
        pltpu.VMEM((CHUNK,), jnp.float32),
        pltpu.VMEM((RPS,), jnp.float32),
        pltpu.VMEM_SHARED((NACC,), jnp.float32),
    ],
)
def _sc_degrees(dstc_hbm, out_hbm, didx, ones, zbuf, acc):
    c = lax.axis_index("c")
    s = lax.axis_index("s")
    w = c * NS + s
    one16 = jnp.ones((16,), jnp.float32)
    zero16 = jnp.zeros((16,), jnp.float32)
    for l in range(CHUNK // 16):
        ones[pl.ds(16 * l, 16)] = one16

    def zfill(i, _):
        zbuf[pl.ds(16 * i, 16)] = zero16
        return 0
    lax.fori_loop(0, RPS // 16, zfill, 0)
    pltpu.sync_copy(zbuf, acc.at[pl.ds(s * RPS, RPS)])
    plsc.subcore_barrier()

    pltpu.sync_copy(dstc_hbm.at[w], didx)

    def body(j, _):
        pltpu.sync_copy(ones, acc.at[didx.at[j]], add=True)
        return 0
    lax.fori_loop(0, CPW, body, 0)
    plsc.subcore_barrier()
    pltpu.sync_copy(acc.at[pl.ds(s * RPS, RPS)],
                    out_hbm.at[c, pl.ds(s * RPS, RPS)])


@functools.partial(
    pl.kernel,
    out_type=jax.ShapeDtypeStruct((NC, NACC, D), jnp.float32),
    mesh=_mesh,
    scratch_types=[
        pltpu.VMEM((CPW + 3, CHUNK), jnp.int32),
        pltpu.VMEM((CPW + 3, CHUNK), jnp.int32),
        pltpu.VMEM((CHUNK, D), jnp.float32),
        pltpu.VMEM((8, D), jnp.float32),
        pltpu.VMEM_SHARED((NACC, D), jnp.float32),
        pltpu.SemaphoreType.DMA,
    ],
)
def _sc_edge_pass(z_hbm, srcc_hbm, dstc_hbm, out_hbm,
                  sidx, didx, rows, zrow, acc, sem):
    c = lax.axis_index("c")
    s = lax.axis_index("s")
    w = c * NS + s
    zero16 = jnp.zeros((16,), jnp.float32)
    for r in range(8):
        for l in range(D // 16):
            zrow[r, pl.ds(16 * l, 16)] = zero16

    def zfill(i, _):
        pltpu.sync_copy(zrow, acc.at[pl.ds(s * RPS + 8 * i, 8)])
        return 0
    lax.fori_loop(0, RPS // 8, zfill, 0)
    plsc.subcore_barrier()

    pltpu.sync_copy(srcc_hbm.at[w], sidx.at[pl.ds(0, CPW)])
    pltpu.sync_copy(dstc_hbm.at[w], didx.at[pl.ds(0, CPW)])

    def body(j, _):
        pltpu.async_copy(z_hbm.at[sidx.at[j]], rows, sem).wait()
        pltpu.sync_copy(rows, acc.at[didx.at[j]], add=True)
        return 0
    lax.fori_loop(0, CPW, body, 0)
    plsc.subcore_barrier()
    pltpu.sync_copy(acc.at[pl.ds(s * RPS, RPS)],
                    out_hbm.at[c, pl.ds(s * RPS, RPS)])


def _scale_body(x_ref, s_ref, o_ref, ys_ref):
    z = x_ref[...] * s_ref[...]
    o_ref[...] = z
    ys_ref[...] = z


def _combine_body(p0_ref, p1_ref, n2_ref, ys_ref, z_ref, yso_ref):
    z = (p0_ref[...] + p1_ref[...]) * n2_ref[...]
    z_ref[...] = z
    yso_ref[...] = ys_ref[...] + z


def _mlp_body(ys_ref, sq_ref, w1_ref, b1_ref, w2_ref, b2_ref, o_ref):
    y = ys_ref[...] * (sq_ref[...] * (1.0 / (K_PROP + 1)))
    h1 = jnp.maximum(
        jnp.dot(y, w1_ref[...], preferred_element_type=jnp.float32)
        + b1_ref[...], 0.0)
    logits = (jnp.dot(h1, w2_ref[...], preferred_element_type=jnp.float32)
              + b2_ref[...])
    m = jnp.max(logits, axis=-1, keepdims=True)
    lse = jnp.log(jnp.sum(jnp.exp(logits - m), axis=-1, keepdims=True))
    o_ref[...] = logits - m - lse


_RB = 1024


def _tc_scale(x, scal):
    return pl.pallas_call(
        _scale_body,
        grid=(NACC // _RB,),
        in_specs=[pl.BlockSpec((_RB, D), lambda i: (i, 0)),
                  pl.BlockSpec((_RB, 1), lambda i: (i, 0))],
        out_specs=[pl.BlockSpec((_RB, D), lambda i: (i, 0)),
                   pl.BlockSpec((_RB, D), lambda i: (i, 0))],
        out_shape=[jax.ShapeDtypeStruct((NACC, D), jnp.float32),
                   jax.ShapeDtypeStruct((NACC, D), jnp.float32)],
    )(x, scal)


def _tc_combine(p0, p1, n2, ysum):
    return pl.pallas_call(
        _combine_body,
        grid=(NACC // _RB,),
        in_specs=[pl.BlockSpec((_RB, D), lambda i: (i, 0)),
                  pl.BlockSpec((_RB, D), lambda i: (i, 0)),
                  pl.BlockSpec((_RB, 1), lambda i: (i, 0)),
                  pl.BlockSpec((_RB, D), lambda i: (i, 0))],
        out_specs=[pl.BlockSpec((_RB, D), lambda i: (i, 0)),
                   pl.BlockSpec((_RB, D), lambda i: (i, 0))],
        out_shape=[jax.ShapeDtypeStruct((NACC, D), jnp.float32),
                   jax.ShapeDtypeStruct((NACC, D), jnp.float32)],
    )(p0, p1, n2, ysum)


_MB = 1000


def _tc_mlp(ysum, sq, W1, b1, W2, b2):
    return pl.pallas_call(
        _mlp_body,
        grid=(N // _MB,),
        in_specs=[pl.BlockSpec((_MB, D), lambda i: (i, 0)),
                  pl.BlockSpec((_MB, 1), lambda i: (i, 0)),
                  pl.BlockSpec((D, H), lambda i: (0, 0)),
                  pl.BlockSpec((1, H), lambda i: (0, 0)),
                  pl.BlockSpec((H, C), lambda i: (0, 0)),
                  pl.BlockSpec((1, C), lambda i: (0, 0))],
        out_specs=pl.BlockSpec((_MB, C), lambda i: (i, 0)),
        out_shape=jax.ShapeDtypeStruct((N, C), jnp.float32),
    )(ysum, sq, W1, b1.reshape(1, H), W2, b2.reshape(1, C))


def kernel(feats, edge_index, W1, b1, W2, b2):
    src = edge_index[0]
    dst = edge_index[1]
    pad = EPAD - E
    srcc = jnp.concatenate(
        [src, jnp.zeros((pad,), jnp.int32)]).reshape(NW, CPW, CHUNK)
    pad_dst = jnp.full((pad,), N, jnp.int32)
    dstc = jnp.concatenate([dst, pad_dst]).reshape(NW, CPW, CHUNK)
    featsp = jnp.concatenate(
        [feats, jnp.zeros((NACC - N, D), jnp.float32)], axis=0)

    deg_part = _sc_degrees(dstc)
    deg = jnp.maximum(deg_part[0] + deg_part[1], 1.0)
    n1 = lax.rsqrt(deg)[:, None]
    n2 = (1.0 / deg)[:, None]
    sq = jnp.sqrt(deg)[:, None]

    z, ysum = _tc_scale(featsp, n1)
    for _ in range(K_PROP):
        p = _sc_edge_pass(z, srcc, dstc)
        z, ysum = _tc_combine(p[0], p[1], n2, ysum)

    return _tc_mlp(ysum, sq, W1, b1, W2, b2)


# speedup vs baseline: 2.7747x; 2.7739x over previous
import functools

import jax
import jax.numpy as jnp
from jax import lax
from jax.experimental import pallas as pl
from jax.experimental.pallas import tpu as pltpu
from jax.experimental.pallas import tpu_sc as plsc

N = 10000
E = 320000
D = 128
H = 256
C = 40
K_PROP = 3

NC = 2
NS = 16
NW = NC * NS
CHUNK = 128
CPW = 80
GPG = 8
NCHUNKS = NW * CPW
EPAD = NCHUNKS * CHUNK
NACC = 10240
RPS = NACC // NS

_mesh = plsc.VectorSubcoreMesh(
    core_axis_name="c", subcore_axis_name="s", num_cores=NC, num_subcores=NS)


@functools.partial(
    pl.kernel,
    out_type=jax.ShapeDtypeStruct((NC, NACC), jnp.float32),
    mesh=_mesh,
    scratch_types=[
        pltpu.VMEM((CPW, CHUNK), jnp.int32),
        pltpu.VMEM((CHUNK,), jnp.float32),
        pltpu.VMEM((RPS,), jnp.float32),
        pltpu.VMEM_SHARED((NACC,), jnp.float32),
    ],
)
def _sc_degrees(dstc_hbm, out_hbm, didx, ones, zbuf, acc):
    c = lax.axis_index("c")
    s = lax.axis_index("s")
    w = c * NS + s
    one16 = jnp.ones((16,), jnp.float32)
    zero16 = jnp.zeros((16,), jnp.float32)
    for l in range(CHUNK // 16):
        ones[pl.ds(16 * l, 16)] = one16

    def zfill(i, _):
        zbuf[pl.ds(16 * i, 16)] = zero16
        return 0
    lax.fori_loop(0, RPS // 16, zfill, 0)
    pltpu.sync_copy(zbuf, acc.at[pl.ds(s * RPS, RPS)])
    plsc.subcore_barrier()

    pltpu.sync_copy(dstc_hbm.at[w], didx)

    def body(j, _):
        pltpu.sync_copy(ones, acc.at[didx.at[j]], add=True)
        return 0
    lax.fori_loop(0, CPW, body, 0)
    plsc.subcore_barrier()
    pltpu.sync_copy(acc.at[pl.ds(s * RPS, RPS)],
                    out_hbm.at[c, pl.ds(s * RPS, RPS)])


@functools.partial(
    pl.kernel,
    out_type=jax.ShapeDtypeStruct((NC, NACC, D), jnp.float32),
    mesh=_mesh,
    scratch_types=[
        pltpu.VMEM((CPW + 3, CHUNK), jnp.int32),
        pltpu.VMEM((CPW + 3, CHUNK), jnp.int32),
        pltpu.VMEM((CHUNK, D), jnp.float32),
        pltpu.VMEM((8, D), jnp.float32),
        pltpu.VMEM_SHARED((NACC, D), jnp.float32),
        pltpu.SemaphoreType.DMA,
    ],
)
def _sc_edge_pass(z_hbm, srcc_hbm, dstc_hbm, out_hbm,
                  sidx, didx, rows, zrow, acc, sem):
    c = lax.axis_index("c")
    s = lax.axis_index("s")
    w = c * NS + s
    zero16 = jnp.zeros((16,), jnp.float32)
    for r in range(8):
        for l in range(D // 16):
            zrow[r, pl.ds(16 * l, 16)] = zero16

    def zfill(i, _):
        pltpu.sync_copy(zrow, acc.at[pl.ds(s * RPS + 8 * i, 8)])
        return 0
    lax.fori_loop(0, RPS // 8, zfill, 0)
    plsc.subcore_barrier()

    pltpu.sync_copy(srcc_hbm.at[w], sidx.at[pl.ds(0, CPW)])
    pltpu.sync_copy(dstc_hbm.at[w], didx.at[pl.ds(0, CPW)])

    def body(j, _):
        pltpu.async_copy(z_hbm.at[sidx.at[j]], rows, sem).wait()
        pltpu.sync_copy(rows, acc.at[didx.at[j]], add=True)
        return 0
    lax.fori_loop(0, CPW, body, 0)
    plsc.subcore_barrier()
    pltpu.sync_copy(acc.at[pl.ds(s * RPS, RPS)],
                    out_hbm.at[c, pl.ds(s * RPS, RPS)])


def _scale_body(x_ref, s_ref, o_ref, ys_ref):
    z = x_ref[...] * s_ref[...]
    o_ref[...] = z
    ys_ref[...] = z


def _combine_body(p0_ref, p1_ref, n2_ref, ys_ref, z_ref, yso_ref):
    z = (p0_ref[...] + p1_ref[...]) * n2_ref[...]
    z_ref[...] = z
    yso_ref[...] = ys_ref[...] + z


def _mlp_body(ys_ref, sq_ref, w1_ref, b1_ref, w2_ref, b2_ref, o_ref):
    y = ys_ref[...] * (sq_ref[...] * (1.0 / (K_PROP + 1)))
    h1 = jnp.maximum(
        jnp.dot(y, w1_ref[...], preferred_element_type=jnp.float32)
        + b1_ref[...], 0.0)
    logits = (jnp.dot(h1, w2_ref[...], preferred_element_type=jnp.float32)
              + b2_ref[...])
    m = jnp.max(logits, axis=-1, keepdims=True)
    lse = jnp.log(jnp.sum(jnp.exp(logits - m), axis=-1, keepdims=True))
    o_ref[...] = logits - m - lse


_RB = 1024


def _tc_scale(x, scal):
    return pl.pallas_call(
        _scale_body,
        grid=(NACC // _RB,),
        in_specs=[pl.BlockSpec((_RB, D), lambda i: (i, 0)),
                  pl.BlockSpec((_RB, 1), lambda i: (i, 0))],
        out_specs=[pl.BlockSpec((_RB, D), lambda i: (i, 0)),
                   pl.BlockSpec((_RB, D), lambda i: (i, 0))],
        out_shape=[jax.ShapeDtypeStruct((NACC, D), jnp.float32),
                   jax.ShapeDtypeStruct((NACC, D), jnp.float32)],
    )(x, scal)


def _tc_combine(p0, p1, n2, ysum):
    return pl.pallas_call(
        _combine_body,
        grid=(NACC // _RB,),
        in_specs=[pl.BlockSpec((_RB, D), lambda i: (i, 0)),
                  pl.BlockSpec((_RB, D), lambda i: (i, 0)),
                  pl.BlockSpec((_RB, 1), lambda i: (i, 0)),
                  pl.BlockSpec((_RB, D), lambda i: (i, 0))],
        out_specs=[pl.BlockSpec((_RB, D), lambda i: (i, 0)),
                   pl.BlockSpec((_RB, D), lambda i: (i, 0))],
        out_shape=[jax.ShapeDtypeStruct((NACC, D), jnp.float32),
                   jax.ShapeDtypeStruct((NACC, D), jnp.float32)],
    )(p0, p1, n2, ysum)


_MB = 1000


def _tc_mlp(ysum, sq, W1, b1, W2, b2):
    return pl.pallas_call(
        _mlp_body,
        grid=(N // _MB,),
        in_specs=[pl.BlockSpec((_MB, D), lambda i: (i, 0)),
                  pl.BlockSpec((_MB, 1), lambda i: (i, 0)),
                  pl.BlockSpec((D, H), lambda i: (0, 0)),
                  pl.BlockSpec((1, H), lambda i: (0, 0)),
                  pl.BlockSpec((H, C), lambda i: (0, 0)),
                  pl.BlockSpec((1, C), lambda i: (0, 0))],
        out_specs=pl.BlockSpec((_MB, C), lambda i: (i, 0)),
        out_shape=jax.ShapeDtypeStruct((N, C), jnp.float32),
    )(ysum, sq, W1, b1.reshape(1, H), W2, b2.reshape(1, C))


def kernel(feats, edge_index, W1, b1, W2, b2):
    src = edge_index[0]
    dst = edge_index[1]
    pad = EPAD - E
    pad_src = jnp.arange(pad, dtype=jnp.int32) % N
    srcc = jnp.concatenate([src, pad_src]).reshape(NW, CPW, CHUNK)
    pad_dst = N + (jnp.arange(pad, dtype=jnp.int32) % (NACC - N))
    dstc = jnp.concatenate([dst, pad_dst]).reshape(NW, CPW, CHUNK)
    featsp = jnp.concatenate(
        [feats, jnp.zeros((NACC - N, D), jnp.float32)], axis=0)

    deg_part = _sc_degrees(dstc)
    deg = jnp.maximum(deg_part[0] + deg_part[1], 1.0)
    n1 = lax.rsqrt(deg)[:, None]
    n2 = (1.0 / deg)[:, None]
    sq = jnp.sqrt(deg)[:, None]

    z, ysum = _tc_scale(featsp, n1)
    for _ in range(K_PROP):
        p = _sc_edge_pass(z, srcc, dstc)
        z, ysum = _tc_combine(p[0], p[1], n2, ysum)

    return _tc_mlp(ysum, sq, W1, b1, W2, b2)
